# Initial kernel scaffold; baseline (speedup 1.0000x reference)
import jax
import jax.numpy as jnp
from jax.experimental import pallas as pl


def kernel(z, para, label, persist, sample_edge, ar_pair, Aw_idx, Aw_val, Ab_idx, Ab_val, new_idx):
    raise NotImplementedError("write your pallas kernel here")


# speedup vs baseline: 6.2755x; 6.2755x over previous
import functools

import jax
import jax.numpy as jnp
from jax import lax
from jax.experimental import pallas as pl
from jax.experimental.pallas import tpu as pltpu
from jax.experimental.pallas import tpu_sc as plsc

N = 5000
T = 20
NT = N * T
M = (T - 1) * N
E = 2000000
P = 80000
NNZ = 3000000
NW = 32

A_CHUNK = 1000
A_NCHUNK = NNZ // A_CHUNK
A_ITERS = -(-A_NCHUNK // NW)
E_CHUNK = 1000
E_NCHUNK = E // E_CHUNK
E_ITERS = -(-E_NCHUNK // NW)
P_CHUNK = 500
P_ITERS = (P // P_CHUNK) // NW

M_PAD = 95232
TPS2 = 2 * M_PAD // 16

_i16 = functools.partial(lax.iota, jnp.int32)


def _f32(x):
    return x.astype(jnp.float32)


def _splat_i(v):
    return jnp.broadcast_to(v, (16,)).astype(jnp.int32)


def _rsqrt(x):
    b = plsc.bitcast(x, jnp.int32)
    y = plsc.bitcast(jnp.int32(0x5F3759DF) - (b >> 1), jnp.float32)
    for _ in range(3):
        y = y * (1.5 - 0.5 * x * y * y)
    return y


def _softplus(u):
    e = jnp.exp(jnp.minimum(u, 30.0))
    t = 1.0 + e
    b = plsc.bitcast(t, jnp.int32)
    ex = _f32((b >> 23) - 127)
    m = plsc.bitcast((b & 0x7FFFFF) | 0x3F800000, jnp.float32)
    y = (m - 1.0) / (m + 1.0)
    y2 = y * y
    lnm = y * (2.0 + y2 * (2.0 / 3.0 + y2 * (0.4 + y2 * (2.0 / 7.0))))
    lnt = 0.6931471805599453 * ex + lnm
    return jnp.where(u > 29.0, u, lnt)


def _att_body(z_h, awi_h, awv_h, abi_h, abv_h, para_h, zeros_h,
              out0, out1, att_sh, i0_1d, i1_1d, v_1d,
              sxi_v, syi_v, gxi_v, gyi_v, gx_v, gy_v, sx_v, sy_v,
              para_v, sem):
    cid = lax.axis_index("c")
    sid = lax.axis_index("s")
    w = sid * 2 + cid

    pltpu.sync_copy(para_h, para_v)
    pltpu.sync_copy(zeros_h, att_sh.at[pl.ds(sid * TPS2, TPS2)])

    lane = _i16(16)
    zi = _splat_i(0)
    zf = jnp.zeros((16,), jnp.float32)

    i0_1d[pl.ds(992, 16)] = zi
    i0_1d[pl.ds(1008, 16)] = zi
    i1_1d[pl.ds(992, 16)] = zi
    i1_1d[pl.ds(1008, 16)] = zi
    v_1d[pl.ds(992, 16)] = zf
    v_1d[pl.ds(1008, 16)] = zf
    plsc.subcore_barrier()

    pv = para_v[...]
    for mtx in range(2):
        gamma0 = jnp.broadcast_to(pv[3] if mtx == 0 else pv[4], (16,))
        idx_h = awi_h if mtx == 0 else abi_h
        val_h = awv_h if mtx == 0 else abv_h

        def chunk_body(t, _, idx_h=idx_h, val_h=val_h, gamma0=gamma0):
            g_u = w + NW * t
            validf = jnp.where(_splat_i(g_u) < A_NCHUNK, 1.0, 0.0)
            g = jnp.minimum(g_u, A_NCHUNK - 1)
            start = g * A_CHUNK
            pltpu.sync_copy(idx_h.at[0].at[pl.ds(start, A_CHUNK)],
                            i0_1d.at[pl.ds(0, A_CHUNK)])
            pltpu.sync_copy(idx_h.at[1].at[pl.ds(start, A_CHUNK)],
                            i1_1d.at[pl.ds(0, A_CHUNK)])
            pltpu.sync_copy(val_h.at[pl.ds(start, A_CHUNK)],
                            v_1d.at[pl.ds(0, A_CHUNK)])
            gamma = gamma0 * validf

            def repack(k, _):
                jj = k >> 3
                cc = (k & 7) * 16
                i0x2 = i0_1d[pl.ds(k * 16, 16)] * 2
                i1x2 = i1_1d[pl.ds(k * 16, 16)] * 2
                sxi_v[jj, pl.ds(cc, 16)] = i0x2
                syi_v[jj, pl.ds(cc, 16)] = i0x2 + 1
                gxi_v[jj, pl.ds(cc, 16)] = i1x2
                gyi_v[jj, pl.ds(cc, 16)] = i1x2 + 1
                return 0

            lax.fori_loop(0, 64, repack, 0)
            descs = [pltpu.async_copy(z_h.at[gxi_v.at[j]],
                                      gx_v.at[pl.ds(j * 128, 128)], sem)
                     for j in range(8)]
            descs += [pltpu.async_copy(z_h.at[gyi_v.at[j]],
                                       gy_v.at[pl.ds(j * 128, 128)], sem)
                      for j in range(8)]
            for d in descs:
                d.wait()

            def scale_body(k, _):
                jj = k >> 3
                cc = (k & 7) * 16
                gv = gamma * v_1d[pl.ds(k * 16, 16)]
                sx_v[jj, pl.ds(cc, 16)] = gv * gx_v[pl.ds(k * 16, 16)]
                sy_v[jj, pl.ds(cc, 16)] = gv * gy_v[pl.ds(k * 16, 16)]
                return 0

            lax.fori_loop(0, 64, scale_body, 0)
            descs2 = [pltpu.async_copy(sx_v.at[j],
                                       att_sh.at[sxi_v.at[j]], sem, add=True)
                      for j in range(8)]
            descs2 += [pltpu.async_copy(sy_v.at[j],
                                        att_sh.at[syi_v.at[j]], sem, add=True)
                       for j in range(8)]
            for d in descs2:
                d.wait()
            return 0

        lax.fori_loop(0, A_ITERS, chunk_body, 0)

    plsc.subcore_barrier()
    off = sid * TPS2

    @pl.when(cid == 0)
    def _():
        pltpu.sync_copy(att_sh.at[pl.ds(off, TPS2)], out0.at[pl.ds(off, TPS2)])

    @pl.when(cid == 1)
    def _():
        pltpu.sync_copy(att_sh.at[pl.ds(off, TPS2)], out1.at[pl.ds(off, TPS2)])


@functools.lru_cache(maxsize=1)
def _att_kernel():
    mesh = plsc.VectorSubcoreMesh(
        core_axis_name="c", subcore_axis_name="s", num_cores=2,
        num_subcores=16)
    return pl.kernel(
        _att_body,
        compiler_params=pltpu.CompilerParams(
            needs_layout_passes=False, use_tc_tiling_on_sc=False),
        out_type=[jax.ShapeDtypeStruct((2 * M_PAD,), jnp.float32),
                  jax.ShapeDtypeStruct((2 * M_PAD,), jnp.float32)],
        mesh=mesh,
        scratch_types=[
            pltpu.MemorySpace.VMEM_SHARED((2 * M_PAD,), jnp.float32),
            pltpu.MemorySpace.VMEM((1024,), jnp.int32),
            pltpu.MemorySpace.VMEM((1024,), jnp.int32),
            pltpu.MemorySpace.VMEM((1024,), jnp.float32),
            pltpu.MemorySpace.VMEM((8, 128), jnp.int32),
            pltpu.MemorySpace.VMEM((8, 128), jnp.int32),
            pltpu.MemorySpace.VMEM((8, 128), jnp.int32),
            pltpu.MemorySpace.VMEM((8, 128), jnp.int32),
            pltpu.MemorySpace.VMEM((1024,), jnp.float32),
            pltpu.MemorySpace.VMEM((1024,), jnp.float32),
            pltpu.MemorySpace.VMEM((8, 128), jnp.float32),
            pltpu.MemorySpace.VMEM((8, 128), jnp.float32),
            pltpu.MemorySpace.VMEM((16,), jnp.float32),
            pltpu.SemaphoreType.DMA,
        ],
    )


def _main_body(z_h, para_h, se_h, lb_h, ps_h, ar_h, att0_h, att1_h, ni_h,
               zeros_h, out,
               e_v, txi_v, tyi_v, sxi_v, syi_v, lb_v, ps_v,
               tx_v, ty_v, sx_v, sy_v,
               ar_v, ia_v, ib_v, ic_v, id_v,
               zax_v, zay_v, zbx_v, zby_v, a0x_v, a0y_v, a1x_v, a1y_v,
               blk_v, ni_v, nia_v, nib_v, nrx_v, nry_v, p2_v, para_v,
               part_v, sem):
    cid = lax.axis_index("c")
    sid = lax.axis_index("s")
    w = sid * 2 + cid

    pltpu.sync_copy(para_h, para_v)
    pv = para_v[...]
    alpha = jnp.broadcast_to(pv[1], (16,))
    delta = jnp.broadcast_to(pv[5], (16,))

    lane = _i16(16)
    half = lane >> 1
    parity = lane & 1
    zeros16 = _splat_i(0)
    ones16 = _splat_i(1)
    evenb = parity == 0
    zf = jnp.zeros((16,), jnp.float32)
    acc = jnp.zeros((16,), jnp.float32)

    lb_v[pl.ds(992, 16)] = zf
    lb_v[pl.ds(1008, 16)] = zf
    ps_v[pl.ds(992, 16)] = zf
    ps_v[pl.ds(1008, 16)] = zf

    def edge_chunk(t, acc):
        g_u = w + NW * t
        validf = jnp.where(_splat_i(g_u) < E_NCHUNK, 1.0, 0.0)
        g = jnp.minimum(g_u, E_NCHUNK - 1)
        row0 = g * E_CHUNK
        pltpu.sync_copy(se_h.at[pl.ds(row0, E_CHUNK)],
                        e_v.at[pl.ds(0, E_CHUNK)])
        pltpu.sync_copy(lb_h.at[pl.ds(row0, E_CHUNK)],
                        lb_v.at[pl.ds(0, E_CHUNK)])
        pltpu.sync_copy(ps_h.at[pl.ds(row0, E_CHUNK)],
                        ps_v.at[pl.ds(0, E_CHUNK)])

        def extract(k, _):
            jj = k >> 3
            cc = (k & 7) * 16
            rr = jnp.minimum(k * 16 + lane, E_CHUNK - 1)
            ti2 = plsc.load_gather(e_v, [rr, zeros16]) * 2
            si2 = plsc.load_gather(e_v, [rr, ones16]) * 2
            txi_v[jj, pl.ds(cc, 16)] = ti2
            tyi_v[jj, pl.ds(cc, 16)] = ti2 + 1
            sxi_v[jj, pl.ds(cc, 16)] = si2
            syi_v[jj, pl.ds(cc, 16)] = si2 + 1
            return 0

        lax.fori_loop(0, 64, extract, 0)
        descs = [pltpu.async_copy(z_h.at[ii.at[j]],
                                  dd.at[pl.ds(j * 128, 128)], sem)
                 for ii, dd in ((txi_v, tx_v), (tyi_v, ty_v),
                                (sxi_v, sx_v), (syi_v, sy_v))
                 for j in range(8)]
        for d in descs:
            d.wait()

        def edge_math(k, acc):
            sl = pl.ds(k * 16, 16)
            dx = tx_v[sl] - sx_v[sl]
            dy = ty_v[sl] - sy_v[sl]
            d2 = dx * dx + dy * dy + 1e-12
            dist = d2 * _rsqrt(d2)
            lb = lb_v[sl]
            ps = ps_v[sl]
            eta = alpha - dist + delta * ps
            eta_c = jnp.clip(eta, -90.0, 15.0)
            mid = lb * eta_c - _softplus(eta_c)
            logp = jnp.where(eta > 15.0, (1.0 - lb) * (-eta),
                             jnp.where(eta < -90.0, lb * eta, mid))
            mask = jnp.where(k * 16 + lane < E_CHUNK, 1.0, 0.0) * validf
            return acc + mask * logp

        return lax.fori_loop(0, 64, edge_math, acc)

    acc = lax.fori_loop(0, E_ITERS, edge_chunk, acc)

    def ar_chunk(t, acc):
        row0 = (w + NW * t) * P_CHUNK
        pltpu.sync_copy(ar_h.at[pl.ds(row0, P_CHUNK)],
                        ar_v.at[pl.ds(0, P_CHUNK)])

        def extract(k, _):
            jj = k >> 3
            cc = (k & 7) * 16
            rr = jnp.minimum(k * 16 + lane, P_CHUNK - 1)
            s2 = plsc.load_gather(ar_v, [rr, zeros16]) * 2
            s2b = plsc.load_gather(ar_v, [rr, ones16]) * 2
            ia_v[jj, pl.ds(cc, 16)] = s2
            ib_v[jj, pl.ds(cc, 16)] = s2 + 1
            ic_v[jj, pl.ds(cc, 16)] = s2b
            id_v[jj, pl.ds(cc, 16)] = s2b + 1
            return 0

        lax.fori_loop(0, 32, extract, 0)
        descs = [pltpu.async_copy(tt.at[ii.at[j]],
                                  dd.at[pl.ds(j * 128, 128)], sem)
                 for tt, ii, dd in
                 ((z_h, ia_v, zax_v), (z_h, ib_v, zay_v),
                  (z_h, ic_v, zbx_v), (z_h, id_v, zby_v),
                  (att0_h, ia_v, a0x_v), (att0_h, ib_v, a0y_v),
                  (att1_h, ia_v, a1x_v), (att1_h, ib_v, a1y_v))
                 for j in range(4)]
        for d in descs:
            d.wait()

        def ar_math(k, acc):
            sl = pl.ds(k * 16, 16)
            wx = zbx_v[sl] - zax_v[sl] - a0x_v[sl] - a1x_v[sl]
            wy = zby_v[sl] - zay_v[sl] - a0y_v[sl] - a1y_v[sl]
            mask = jnp.where(k * 16 + lane < P_CHUNK, 1.0, 0.0)
            return acc + mask * (-0.5 * (wx * wx + wy * wy))

        return lax.fori_loop(0, 32, ar_math, acc)

    acc = lax.fori_loop(0, P_ITERS, ar_chunk, acc)

    pltpu.sync_copy(z_h.at[pl.ds(w * 320, 320)], p2_v)

    def p2_math(k, acc):
        v = p2_v[pl.ds(k * 16, 16)]
        gid = w * 320 + k * 16 + lane
        mask = jnp.where(gid < 2 * N, 1.0, 0.0)
        return acc + mask * (-0.5 * v * v)

    acc = lax.fori_loop(0, 20, p2_math, acc)

    pltpu.sync_copy(zeros_h.at[pl.ds(0, 5040)], blk_v)
    evenf = jnp.where(evenb, 1.0, 0.0)
    oddf = 1.0 - evenf

    def p4_group(gi, acc):
        g = w + NW * gi
        validf = jnp.where(_splat_i(g) < 38, 1.0, 0.0)
        g_eff = jnp.minimum(g, 37)
        pltpu.sync_copy(z_h.at[pl.ds(g_eff * 5000, 5000)],
                        blk_v.at[pl.ds(0, 5000)])
        pltpu.sync_copy(ni_h.at[g_eff], ni_v)

        def mkidx(k, _):
            n2 = ni_v[pl.ds(k * 16, 16)] * 2
            nia_v[pl.ds(k * 16, 16)] = n2
            nib_v[pl.ds(k * 16, 16)] = n2 + 1
            return 0

        lax.fori_loop(0, 8, mkidx, 0)
        pltpu.async_copy(z_h.at[nia_v], nrx_v, sem).wait()
        pltpu.async_copy(z_h.at[nib_v], nry_v, sem).wait()

        def blk_sum(k, bs):
            return bs + blk_v[pl.ds(k * 16, 16)]

        bs = lax.fori_loop(0, 315, blk_sum, jnp.zeros((16,), jnp.float32))
        sx = jnp.sum(bs * evenf)
        sy = jnp.sum(bs * oddf)
        mx = jnp.broadcast_to(sx * (1.0 / 2500.0), (16,))
        my = jnp.broadcast_to(sy * (1.0 / 2500.0), (16,))

        def new_math(k, acc):
            sl = pl.ds(k * 16, 16)
            ddx = nrx_v[sl] - mx
            ddy = nry_v[sl] - my
            mask = jnp.where(k * 16 + lane < 125, 1.0, 0.0) * validf
            return acc + mask * (-0.5 * (ddx * ddx + ddy * ddy))

        return lax.fori_loop(0, 8, new_math, acc)

    acc = lax.fori_loop(0, 2, p4_group, acc)

    part_v[...] = acc
    pltpu.sync_copy(part_v, out.at[w])


@functools.lru_cache(maxsize=1)
def _main_kernel():
    mesh = plsc.VectorSubcoreMesh(
        core_axis_name="c", subcore_axis_name="s", num_cores=2,
        num_subcores=16)
    return pl.kernel(
        _main_body,
        compiler_params=pltpu.CompilerParams(
            needs_layout_passes=False, use_tc_tiling_on_sc=False),
        out_type=jax.ShapeDtypeStruct((NW, 16), jnp.float32),
        mesh=mesh,
        scratch_types=[
            pltpu.MemorySpace.VMEM((1024, 2), jnp.int32),
            pltpu.MemorySpace.VMEM((8, 128), jnp.int32),
            pltpu.MemorySpace.VMEM((8, 128), jnp.int32),
            pltpu.MemorySpace.VMEM((8, 128), jnp.int32),
            pltpu.MemorySpace.VMEM((8, 128), jnp.int32),
            pltpu.MemorySpace.VMEM((1024,), jnp.float32),
            pltpu.MemorySpace.VMEM((1024,), jnp.float32),
            pltpu.MemorySpace.VMEM((1024,), jnp.float32),
            pltpu.MemorySpace.VMEM((1024,), jnp.float32),
            pltpu.MemorySpace.VMEM((1024,), jnp.float32),
            pltpu.MemorySpace.VMEM((1024,), jnp.float32),
            pltpu.MemorySpace.VMEM((512, 2), jnp.int32),
            pltpu.MemorySpace.VMEM((4, 128), jnp.int32),
            pltpu.MemorySpace.VMEM((4, 128), jnp.int32),
            pltpu.MemorySpace.VMEM((4, 128), jnp.int32),
            pltpu.MemorySpace.VMEM((4, 128), jnp.int32),
            pltpu.MemorySpace.VMEM((512,), jnp.float32),
            pltpu.MemorySpace.VMEM((512,), jnp.float32),
            pltpu.MemorySpace.VMEM((512,), jnp.float32),
            pltpu.MemorySpace.VMEM((512,), jnp.float32),
            pltpu.MemorySpace.VMEM((512,), jnp.float32),
            pltpu.MemorySpace.VMEM((512,), jnp.float32),
            pltpu.MemorySpace.VMEM((512,), jnp.float32),
            pltpu.MemorySpace.VMEM((512,), jnp.float32),
            pltpu.MemorySpace.VMEM((5040,), jnp.float32),
            pltpu.MemorySpace.VMEM((128,), jnp.int32),
            pltpu.MemorySpace.VMEM((128,), jnp.int32),
            pltpu.MemorySpace.VMEM((128,), jnp.int32),
            pltpu.MemorySpace.VMEM((128,), jnp.float32),
            pltpu.MemorySpace.VMEM((128,), jnp.float32),
            pltpu.MemorySpace.VMEM((320,), jnp.float32),
            pltpu.MemorySpace.VMEM((16,), jnp.float32),
            pltpu.MemorySpace.VMEM((16,), jnp.float32),
            pltpu.SemaphoreType.DMA,
        ],
    )


def _pad1(x, n, val=0):
    return jnp.concatenate(
        [x, jnp.full((n - x.shape[0],) + x.shape[1:], val, x.dtype)])


def kernel(z, para, label, persist, sample_edge, ar_pair,
           Aw_idx, Aw_val, Ab_idx, Ab_val, new_idx):
    z1d = z.astype(jnp.float32).reshape(2 * NT)
    se = sample_edge.astype(jnp.int32)
    ar = ar_pair.astype(jnp.int32)
    ni = _pad1(new_idx.astype(jnp.int32).reshape(38, 125).T, 128).T
    zeros_blk = jnp.zeros((TPS2,), jnp.float32)
    para_flat = _pad1(para.astype(jnp.float32).reshape(6), 16)

    att0, att1 = _att_kernel()(z1d, Aw_idx.astype(jnp.int32),
                               Aw_val.astype(jnp.float32),
                               Ab_idx.astype(jnp.int32),
                               Ab_val.astype(jnp.float32),
                               para_flat, zeros_blk)
    part = _main_kernel()(z1d, para_flat, se, label.astype(jnp.float32),
                          persist.astype(jnp.float32), ar, att0, att1, ni,
                          zeros_blk)
    return -jnp.sum(part)
